# baseline (speedup 1.0000x reference)
import functools

import jax
import jax.numpy as jnp
from jax import lax
from jax.experimental import pallas as pl
from jax.experimental.pallas import tpu as pltpu
from jax.experimental.pallas import tpu_sc as plsc

_D = 64
_NC = 2
_NS = 16
_NW = _NC * _NS
_CH = 128


def _sc_gather(x_flat, table_bf):
    batch = x_flat.shape[0]
    b_per_w = batch // _NW
    n_ch = b_per_w // _CH
    mesh = plsc.VectorSubcoreMesh(core_axis_name="c", subcore_axis_name="s")

    @functools.partial(
        pl.kernel,
        mesh=mesh,
        out_type=jax.ShapeDtypeStruct((batch, _D), jnp.bfloat16),
        scratch_types=[
            pltpu.VMEM((b_per_w,), jnp.int32),
            pltpu.VMEM((_CH, _D), jnp.bfloat16),
            pltpu.SemaphoreType.DMA,
        ],
        compiler_params=pltpu.CompilerParams(use_tc_tiling_on_sc=False),
    )
    def gather_kernel(idx_hbm, table_hbm, out_hbm, idx_v, rows_v, sem):
        wid = lax.axis_index("s") * _NC + lax.axis_index("c")
        base = wid * b_per_w
        pltpu.sync_copy(idx_hbm.at[pl.ds(base, b_per_w)], idx_v)

        def body(j, carry):
            pltpu.async_copy(
                table_hbm.at[idx_v.at[pl.ds(j * _CH, _CH)]], rows_v, sem
            ).wait()
            pltpu.sync_copy(rows_v, out_hbm.at[pl.ds(base + j * _CH, _CH)])
            return carry

        lax.fori_loop(0, n_ch, body, 0)

    return gather_kernel(x_flat, table_bf)


def _tc_project_1d(rows_1d, w_t, b2):
    m = rows_1d.shape[0] // _D
    tm = 16384

    def mm(g_ref, w_ref, b_ref, o_ref):
        g2 = g_ref[...].reshape(tm // 2, 2 * _D)
        w = w_ref[...]
        ol = jnp.dot(g2[:, :_D], w, preferred_element_type=jnp.float32)
        orr = jnp.dot(g2[:, _D:], w, preferred_element_type=jnp.float32)
        o_ref[...] = jnp.stack([ol, orr], axis=1).reshape(tm, _D) + b_ref[...]

    return pl.pallas_call(
        mm,
        grid=(m // tm,),
        in_specs=[
            pl.BlockSpec((tm * _D,), lambda i: (i,)),
            pl.BlockSpec((_D, _D), lambda i: (0, 0)),
            pl.BlockSpec((1, _D), lambda i: (0, 0)),
        ],
        out_specs=pl.BlockSpec((tm, _D), lambda i: (i, 0)),
        out_shape=jax.ShapeDtypeStruct((m, _D), jnp.float32),
    )(rows_1d, w_t, b2)


def kernel(x, encodings, W, b):
    num_paths, path_len = x.shape
    batch = num_paths * path_len
    x_flat = x.reshape(-1).astype(jnp.int32)
    table_bf = encodings.astype(jnp.bfloat16)
    gathered = _sc_gather(x_flat, table_bf)
    out = _tc_project_1d(
        gathered.reshape(-1), W.T.astype(jnp.bfloat16), b.reshape(1, _D)
    )
    return out


# speedup vs baseline: 1.3326x; 1.3326x over previous
import functools

import jax
import jax.numpy as jnp
from jax import lax
from jax.experimental import pallas as pl
from jax.experimental.pallas import tpu as pltpu
from jax.experimental.pallas import tpu_sc as plsc

_D = 64
_NC = 2
_NS = 16
_NW = _NC * _NS
_CH = 128


def _sc_gather(x_flat, table):
    batch = x_flat.shape[0]
    b_per_w = batch // _NW
    n_ch = b_per_w // _CH
    mesh = plsc.VectorSubcoreMesh(core_axis_name="c", subcore_axis_name="s")

    @functools.partial(
        pl.kernel,
        mesh=mesh,
        out_type=jax.ShapeDtypeStruct((batch, _D), jnp.float32),
        scratch_types=[
            pltpu.VMEM((b_per_w,), jnp.int32),
            pltpu.VMEM((_CH, _D), jnp.float32),
            pltpu.SemaphoreType.DMA,
        ],
        compiler_params=pltpu.CompilerParams(use_tc_tiling_on_sc=False),
    )
    def gather_kernel(idx_hbm, table_hbm, out_hbm, idx_v, rows_v, sem):
        wid = lax.axis_index("s") * _NC + lax.axis_index("c")
        base = wid * b_per_w
        pltpu.sync_copy(idx_hbm.at[pl.ds(base, b_per_w)], idx_v)

        def body(j, carry):
            pltpu.async_copy(
                table_hbm.at[idx_v.at[pl.ds(j * _CH, _CH)]], rows_v, sem
            ).wait()
            pltpu.sync_copy(rows_v, out_hbm.at[pl.ds(base + j * _CH, _CH)])
            return carry

        lax.fori_loop(0, n_ch, body, 0)

    return gather_kernel(x_flat, table)


def _tc_project_1d(rows_1d, w_t, b2):
    m = rows_1d.shape[0] // _D
    tm = 16384

    def mm(g_ref, w_ref, b_ref, o_ref):
        g2 = g_ref[...].reshape(tm // 2, 2 * _D)
        w = w_ref[...]
        ol = jnp.dot(g2[:, :_D], w, preferred_element_type=jnp.float32)
        orr = jnp.dot(g2[:, _D:], w, preferred_element_type=jnp.float32)
        o_ref[...] = jnp.stack([ol, orr], axis=1).reshape(tm, _D) + b_ref[...]

    return pl.pallas_call(
        mm,
        grid=(m // tm,),
        in_specs=[
            pl.BlockSpec((tm * _D,), lambda i: (i,)),
            pl.BlockSpec((_D, _D), lambda i: (0, 0)),
            pl.BlockSpec((1, _D), lambda i: (0, 0)),
        ],
        out_specs=pl.BlockSpec((tm, _D), lambda i: (i, 0)),
        out_shape=jax.ShapeDtypeStruct((m, _D), jnp.float32),
    )(rows_1d, w_t, b2)


def kernel(x, encodings, W, b):
    x_flat = x.reshape(-1).astype(jnp.int32)
    gathered = _sc_gather(x_flat, encodings)
    out = _tc_project_1d(gathered.reshape(-1), W.T, b.reshape(1, _D))
    return out
